# baseline (speedup 1.0000x reference)
import numpy as np
import jax
import jax.numpy as jnp
from jax.experimental import pallas as pl
from jax.experimental.pallas import tpu as pltpu

D = 512
H = 512
CD = 512
L = 128
B = 16
T = 512
TOTAL = 4352
PAD = TOTAL + B
SCALE = 1.0 / np.sqrt(H)


def _qkt_kernel(wq_ref, wk_ref, c2_ref, o_ref):
    kt = jax.lax.dot_general(
        wk_ref[...], c2_ref[...], (((0,), (1,)), ((), ())),
        preferred_element_type=jnp.float32)
    o_ref[...] = (SCALE * jnp.dot(
        wq_ref[...], kt, preferred_element_type=jnp.float32)
                  ).astype(jnp.bfloat16)


def _proj_kernel(a_ref, b_ref, bias_ref, o_ref):
    o_ref[...] = jnp.dot(a_ref[...], b_ref[...],
                         preferred_element_type=jnp.float32) + bias_ref[...]


def _cw_kernel(a_ref, b_ref, o_ref):
    o_ref[...] = jnp.dot(a_ref[...], b_ref[...],
                         preferred_element_type=jnp.float32
                         ).astype(jnp.bfloat16)


def _loop_kernel(x_ref, hm16_ref, hm8_ref, cw_ref, madd_ref, un_ref,
                 out_ref, hf_ref):

    def make_pair(nb):
        win = nb + 8
        hm_ref = hm16_ref if nb == B else hm8_ref
        lane = jax.lax.broadcasted_iota(jnp.int32, (nb, 1), 0)
        roww = jax.lax.broadcasted_iota(jnp.int32, (win, 1), 0)

        def substep(t, off, h):
            b = B - t // 32
            a8 = off // 8 * 8
            d = off - a8
            hm = jnp.dot(h.astype(jnp.bfloat16), hm_ref[...],
                         preferred_element_type=jnp.float32)
            s = hm[:, 0:nb * L] + madd_ref[0:nb, 0:nb * L]
            m = jnp.max(s, axis=-1, keepdims=True)
            e = jnp.exp(s - m)
            w = (e / jnp.sum(e, axis=-1, keepdims=True)).astype(jnp.bfloat16)
            xwin = pltpu.roll(x_ref[pl.ds(a8, win), :], (win - d) % win,
                              axis=0)
            g = xwin[:nb] + jnp.dot(
                w, cw_ref[0:nb * L, :], preferred_element_type=jnp.float32)
            zr = jax.nn.sigmoid(g[:, : 2 * H] + hm[:, nb * L:])
            z = zr[:, :H]
            r = zr[:, H:]
            n = jnp.tanh(g[:, 2 * H:] + jnp.dot(
                (r * h).astype(jnp.bfloat16), un_ref[...],
                preferred_element_type=jnp.float32))
            hn = (1.0 - z) * n + z * h
            hsel = jnp.where(lane < b, hn, h)
            owin = pltpu.roll(
                jnp.concatenate([hsel, jnp.zeros((8, H), jnp.float32)],
                                axis=0), d, axis=0)
            keep = (roww >= d) & (roww < d + nb)
            out_ref[pl.ds(a8, win), :] = jnp.where(
                keep, owin, out_ref[pl.ds(a8, win), :])
            return off + b, hsel

        def quad(it, carry):
            off, h = carry
            off, h = substep(4 * it, off, h)
            off, h = substep(4 * it + 1, off, h)
            off, h = substep(4 * it + 2, off, h)
            off, h = substep(4 * it + 3, off, h)
            return off, h

        return quad

    h0 = jnp.zeros((B, H), jnp.float32)
    off, h = jax.lax.fori_loop(0, T // 8, make_pair(B), (jnp.int32(0), h0))
    hf_ref[0, B // 2:, :] = h[B // 2:]
    _, h8 = jax.lax.fori_loop(T // 8, T // 4, make_pair(B // 2),
                              (off, h[: B // 2]))
    hf_ref[0, 0: B // 2, :] = h8


def kernel(pack_data, batch_sizes, context, context_mask, Wq, Wk, Wz, Wr, Wn,
           Uz, Ur, Un, bz, br, bn):
    f32 = jnp.float32
    pack_pad = jnp.zeros((PAD, D), f32).at[:TOTAL].set(pack_data)
    wcat = jnp.concatenate([Wz[:D], Wr[:D], Wn[:D]], axis=1)
    bcat = jnp.concatenate([bz, br, bn])[None, :]
    wa = jnp.concatenate([Wz[D:], Wr[D:], Wn[D:]], axis=1)
    ucat = jnp.concatenate([Uz, Ur], axis=1)
    ctx2 = context.reshape(B * L, CD)
    madd1 = jnp.where(context_mask, 0.0, -1e9).astype(f32)
    madd = jnp.where(jnp.eye(B, dtype=bool)[:, :, None],
                     madd1[:, None, :], -1e9).reshape(B, B * L).astype(f32)

    QKT = pl.pallas_call(
        _qkt_kernel,
        out_shape=jax.ShapeDtypeStruct((H, B * L), jnp.bfloat16),
    )(Wq, Wk, ctx2)
    ucat_b = ucat.astype(jnp.bfloat16)
    hm16 = jnp.concatenate([QKT, ucat_b], axis=1)
    hm8 = jnp.concatenate([QKT[:, : B * L // 2], ucat_b], axis=1)

    CW = pl.pallas_call(
        _cw_kernel,
        out_shape=jax.ShapeDtypeStruct((B * L, 3 * H), jnp.bfloat16),
    )(ctx2, wa)

    X = pl.pallas_call(
        _proj_kernel,
        out_shape=jax.ShapeDtypeStruct((PAD, 3 * H), f32),
        compiler_params=pltpu.CompilerParams(vmem_limit_bytes=100 * 2**20),
    )(pack_pad, wcat, bcat)

    out_pad, hidden_final = pl.pallas_call(
        _loop_kernel,
        out_shape=(jax.ShapeDtypeStruct((PAD, H), f32),
                   jax.ShapeDtypeStruct((1, B, H), f32)),
        compiler_params=pltpu.CompilerParams(vmem_limit_bytes=110 * 2**20),
    )(X, hm16, hm8, CW, madd, Un.astype(jnp.bfloat16))

    return out_pad[:TOTAL], hidden_final


# speedup vs baseline: 1.1214x; 1.1214x over previous
import numpy as np
import jax
import jax.numpy as jnp
from jax.experimental import pallas as pl
from jax.experimental.pallas import tpu as pltpu

D = 512
H = 512
CD = 512
L = 128
B = 16
T = 512
TOTAL = 4352
PAD = TOTAL + B
SCALE = 1.0 / np.sqrt(H)


def _qkt_kernel(wq_ref, wk_ref, c2_ref, o_ref):
    kt = jax.lax.dot_general(
        wk_ref[...], c2_ref[...], (((0,), (1,)), ((), ())),
        preferred_element_type=jnp.float32)
    o_ref[...] = (SCALE * jnp.dot(
        wq_ref[...], kt, preferred_element_type=jnp.float32)
                  ).astype(jnp.bfloat16)


def _proj_kernel(a_ref, b_ref, bias_ref, o_ref):
    o_ref[...] = jnp.dot(a_ref[...], b_ref[...],
                         preferred_element_type=jnp.float32) + bias_ref[...]


def _loop_kernel(x_ref, hm16_ref, hm12_ref, hm8_ref, hm4_ref, ctx2_ref,
                 madd_ref, un_ref, wa_ref, out_ref, hf_ref):

    def make_quad(nb, ns, hm_ref):
        win = nb + 8
        lane = jax.lax.broadcasted_iota(jnp.int32, (nb, 1), 0)
        roww = jax.lax.broadcasted_iota(jnp.int32, (win, 1), 0)

        def substep(t, off, h):
            b = B - t // 32
            a8 = off // 8 * 8
            d = off - a8
            hm = jnp.dot(h.astype(jnp.bfloat16), hm_ref[...],
                         preferred_element_type=jnp.float32)
            s = hm[:, 0:ns * L] + madd_ref[0:nb, 0:ns * L]
            m = jnp.max(s, axis=-1, keepdims=True)
            e = jnp.exp(s - m)
            w = (e / jnp.sum(e, axis=-1, keepdims=True)).astype(jnp.bfloat16)
            attn = jnp.dot(w, ctx2_ref[0:ns * L, :],
                           preferred_element_type=jnp.float32)
            xwin = pltpu.roll(x_ref[pl.ds(a8, win), :], (win - d) % win,
                              axis=0)
            g = xwin[:nb] + jnp.dot(
                attn.astype(jnp.bfloat16), wa_ref[...],
                preferred_element_type=jnp.float32)
            zr = jax.nn.sigmoid(g[:, : 2 * H] + hm[:, ns * L:])
            z = zr[:, :H]
            r = zr[:, H:]
            n = jnp.tanh(g[:, 2 * H:] + jnp.dot(
                (r * h).astype(jnp.bfloat16), un_ref[...],
                preferred_element_type=jnp.float32))
            hn = (1.0 - z) * n + z * h
            hsel = jnp.where(lane < b, hn, h)
            owin = pltpu.roll(
                jnp.concatenate([hsel, jnp.zeros((8, H), jnp.float32)],
                                axis=0), d, axis=0)
            keep = (roww >= d) & (roww < d + nb)
            out_ref[pl.ds(a8, win), :] = jnp.where(
                keep, owin, out_ref[pl.ds(a8, win), :])
            return off + b, hsel

        def quad(it, carry):
            off, h = carry
            off, h = substep(4 * it, off, h)
            off, h = substep(4 * it + 1, off, h)
            off, h = substep(4 * it + 2, off, h)
            off, h = substep(4 * it + 3, off, h)
            return off, h

        return quad

    h0 = jnp.zeros((B, H), jnp.float32)
    off, h = jax.lax.fori_loop(0, 32, make_quad(16, 16, hm16_ref),
                               (jnp.int32(0), h0))
    off, h = jax.lax.fori_loop(32, 64, make_quad(16, 12, hm12_ref),
                               (off, h))
    hf_ref[0, 8:, :] = h[8:]
    off, h8 = jax.lax.fori_loop(64, 96, make_quad(8, 8, hm8_ref),
                                (off, h[:8]))
    _, h8 = jax.lax.fori_loop(96, 128, make_quad(8, 4, hm4_ref),
                              (off, h8))
    hf_ref[0, 0:8, :] = h8


def kernel(pack_data, batch_sizes, context, context_mask, Wq, Wk, Wz, Wr, Wn,
           Uz, Ur, Un, bz, br, bn):
    f32 = jnp.float32
    pack_pad = jnp.zeros((PAD, D), f32).at[:TOTAL].set(pack_data)
    wcat = jnp.concatenate([Wz[:D], Wr[:D], Wn[:D]], axis=1)
    bcat = jnp.concatenate([bz, br, bn])[None, :]
    wa = jnp.concatenate([Wz[D:], Wr[D:], Wn[D:]], axis=1)
    ucat = jnp.concatenate([Uz, Ur], axis=1)
    ctx2 = context.reshape(B * L, CD)
    madd1 = jnp.where(context_mask, 0.0, -1e9).astype(f32)
    madd = jnp.where(jnp.eye(B, dtype=bool)[:, :, None],
                     madd1[:, None, :], -1e9).reshape(B, B * L).astype(f32)

    QKT = pl.pallas_call(
        _qkt_kernel,
        out_shape=jax.ShapeDtypeStruct((H, B * L), jnp.bfloat16),
    )(Wq, Wk, ctx2)
    ucat_b = ucat.astype(jnp.bfloat16)
    hm16 = jnp.concatenate([QKT, ucat_b], axis=1)
    hm12 = jnp.concatenate([QKT[:, : 12 * L], ucat_b], axis=1)
    hm8 = jnp.concatenate([QKT[:, : 8 * L], ucat_b], axis=1)
    hm4 = jnp.concatenate([QKT[:, : 4 * L], ucat_b], axis=1)

    X = pl.pallas_call(
        _proj_kernel,
        out_shape=jax.ShapeDtypeStruct((PAD, 3 * H), f32),
        compiler_params=pltpu.CompilerParams(vmem_limit_bytes=100 * 2**20),
    )(pack_pad, wcat, bcat)

    out_pad, hidden_final = pl.pallas_call(
        _loop_kernel,
        out_shape=(jax.ShapeDtypeStruct((PAD, H), f32),
                   jax.ShapeDtypeStruct((1, B, H), f32)),
        compiler_params=pltpu.CompilerParams(vmem_limit_bytes=110 * 2**20),
    )(X, hm16, hm12, hm8, hm4, ctx2.astype(jnp.bfloat16), madd,
      Un.astype(jnp.bfloat16), wa.astype(jnp.bfloat16))

    return out_pad[:TOTAL], hidden_final


# speedup vs baseline: 1.2475x; 1.1124x over previous
import numpy as np
import jax
import jax.numpy as jnp
from jax.experimental import pallas as pl
from jax.experimental.pallas import tpu as pltpu

D = 512
H = 512
CD = 512
L = 128
B = 16
T = 512
TOTAL = 4352
PAD = TOTAL + B
SCALE = 1.0 / np.sqrt(H)


def _qkt_kernel(wq_ref, wk_ref, c2_ref, o_ref):
    kt = jax.lax.dot_general(
        wk_ref[...], c2_ref[...], (((0,), (1,)), ((), ())),
        preferred_element_type=jnp.float32)
    o_ref[...] = (SCALE * jnp.dot(
        wq_ref[...], kt, preferred_element_type=jnp.float32)
                  ).astype(jnp.bfloat16)


def _proj_kernel(a_ref, b_ref, bias_ref, o_ref):
    o_ref[...] = jnp.dot(a_ref[...], b_ref[...],
                         preferred_element_type=jnp.float32) + bias_ref[...]


def _loop_kernel(x_ref, hm16_ref, hm12_ref, hm8_ref, hm4_ref, ctx2_ref,
                 madd_ref, un_ref, wa_ref, out_ref, hf_ref):

    def make_quad(nb, ns, hm_ref):
        win = nb + 8
        lane = jax.lax.broadcasted_iota(jnp.int32, (nb, 1), 0)
        roww = jax.lax.broadcasted_iota(jnp.int32, (win, 1), 0)

        def substep(t, off, h):
            b = B - t // 32
            a8 = off // 8 * 8
            d = off - a8
            hm = jnp.dot(h.astype(jnp.bfloat16), hm_ref[...],
                         preferred_element_type=jnp.float32)
            s = hm[:, 0:ns * L] + madd_ref[0:nb, 0:ns * L]
            e = jnp.exp(s)
            recip = 1.0 / jnp.sum(e, axis=-1, keepdims=True)
            attn = jnp.dot(e.astype(jnp.bfloat16), ctx2_ref[0:ns * L, :],
                           preferred_element_type=jnp.float32) * recip
            xwin = pltpu.roll(x_ref[pl.ds(a8, win), :], (win - d) % win,
                              axis=0)
            g = xwin[:nb] + jnp.dot(
                attn.astype(jnp.bfloat16), wa_ref[...],
                preferred_element_type=jnp.float32)
            zr = jax.nn.sigmoid(g[:, : 2 * H] + hm[:, ns * L:])
            z = zr[:, :H]
            r = zr[:, H:]
            n = jnp.tanh(g[:, 2 * H:] + jnp.dot(
                (r * h).astype(jnp.bfloat16), un_ref[...],
                preferred_element_type=jnp.float32))
            hn = (1.0 - z) * n + z * h
            hsel = jnp.where(lane < b, hn, h)
            owin = pltpu.roll(
                jnp.concatenate([hsel, jnp.zeros((8, H), jnp.float32)],
                                axis=0), d, axis=0)
            keep = (roww >= d) & (roww < d + nb)
            out_ref[pl.ds(a8, win), :] = jnp.where(
                keep, owin, out_ref[pl.ds(a8, win), :])
            return off + b, hsel

        def quad(it, carry):
            off, h = carry
            off, h = substep(4 * it, off, h)
            off, h = substep(4 * it + 1, off, h)
            off, h = substep(4 * it + 2, off, h)
            off, h = substep(4 * it + 3, off, h)
            return off, h

        return quad

    h0 = jnp.zeros((B, H), jnp.float32)
    off, h = jax.lax.fori_loop(0, 32, make_quad(16, 16, hm16_ref),
                               (jnp.int32(0), h0))
    off, h = jax.lax.fori_loop(32, 64, make_quad(16, 12, hm12_ref),
                               (off, h))
    hf_ref[0, 8:, :] = h[8:]
    off, h8 = jax.lax.fori_loop(64, 96, make_quad(8, 8, hm8_ref),
                                (off, h[:8]))
    _, h8 = jax.lax.fori_loop(96, 128, make_quad(8, 4, hm4_ref),
                              (off, h8))
    hf_ref[0, 0:8, :] = h8


def kernel(pack_data, batch_sizes, context, context_mask, Wq, Wk, Wz, Wr, Wn,
           Uz, Ur, Un, bz, br, bn):
    f32 = jnp.float32
    pack_pad = jnp.zeros((PAD, D), f32).at[:TOTAL].set(pack_data)
    wcat = jnp.concatenate([Wz[:D], Wr[:D], Wn[:D]], axis=1)
    bcat = jnp.concatenate([bz, br, bn])[None, :]
    wa = jnp.concatenate([Wz[D:], Wr[D:], Wn[D:]], axis=1)
    ucat = jnp.concatenate([Uz, Ur], axis=1)
    ctx2 = context.reshape(B * L, CD)
    madd1 = jnp.where(context_mask, 0.0, -1e9).astype(f32)
    madd = jnp.where(jnp.eye(B, dtype=bool)[:, :, None],
                     madd1[:, None, :], -1e9).reshape(B, B * L).astype(f32)

    QKT = pl.pallas_call(
        _qkt_kernel,
        out_shape=jax.ShapeDtypeStruct((H, B * L), jnp.bfloat16),
    )(Wq, Wk, ctx2)
    ucat_b = ucat.astype(jnp.bfloat16)
    hm16 = jnp.concatenate([QKT, ucat_b], axis=1)
    hm12 = jnp.concatenate([QKT[:, : 12 * L], ucat_b], axis=1)
    hm8 = jnp.concatenate([QKT[:, : 8 * L], ucat_b], axis=1)
    hm4 = jnp.concatenate([QKT[:, : 4 * L], ucat_b], axis=1)

    X = pl.pallas_call(
        _proj_kernel,
        out_shape=jax.ShapeDtypeStruct((PAD, 3 * H), f32),
        compiler_params=pltpu.CompilerParams(vmem_limit_bytes=100 * 2**20),
    )(pack_pad, wcat, bcat)

    out_pad, hidden_final = pl.pallas_call(
        _loop_kernel,
        out_shape=(jax.ShapeDtypeStruct((PAD, H), f32),
                   jax.ShapeDtypeStruct((1, B, H), f32)),
        compiler_params=pltpu.CompilerParams(vmem_limit_bytes=110 * 2**20),
    )(X, hm16, hm12, hm8, hm4, ctx2.astype(jnp.bfloat16), madd,
      Un.astype(jnp.bfloat16), wa.astype(jnp.bfloat16))

    return out_pad[:TOTAL], hidden_final


# speedup vs baseline: 1.2686x; 1.0169x over previous
import numpy as np
import jax
import jax.numpy as jnp
from jax.experimental import pallas as pl
from jax.experimental.pallas import tpu as pltpu

D = 512
H = 512
CD = 512
L = 128
B = 16
T = 512
TOTAL = 4352
PAD = TOTAL + B
SCALE = 1.0 / np.sqrt(H)


def _qkt_kernel(wq_ref, wk_ref, c2_ref, o_ref):
    kt = jax.lax.dot_general(
        wk_ref[...], c2_ref[...], (((0,), (1,)), ((), ())),
        preferred_element_type=jnp.float32)
    o_ref[...] = (SCALE * jnp.dot(
        wq_ref[...], kt, preferred_element_type=jnp.float32)
                  ).astype(jnp.bfloat16)


def _proj_kernel(a_ref, b_ref, bias_ref, o_ref):
    o_ref[...] = jnp.dot(a_ref[...], b_ref[...],
                         preferred_element_type=jnp.float32) + bias_ref[...]


def _loop_kernel(x_ref, hm0_ref, hm1_ref, hm2_ref, hm3_ref, hm4_ref,
                 hm5_ref, hm6_ref, hm7_ref, ctx2_ref, madd_ref, un_ref,
                 wa_ref, out_ref, hf_ref):
    hms = (hm0_ref, hm1_ref, hm2_ref, hm3_ref, hm4_ref, hm5_ref, hm6_ref,
           hm7_ref)

    def make_quad(nb, ns, hm_ref):
        win = nb + 8
        lane = jax.lax.broadcasted_iota(jnp.int32, (nb, 1), 0)
        roww = jax.lax.broadcasted_iota(jnp.int32, (win, 1), 0)

        def substep(t, off, h):
            b = B - t // 32
            a8 = off // 8 * 8
            d = off - a8
            hm = jnp.dot(h.astype(jnp.bfloat16), hm_ref[...],
                         preferred_element_type=jnp.float32)
            s = hm[:, 0:ns * L] + madd_ref[0:nb, 0:ns * L]
            e = jnp.exp(s)
            recip = 1.0 / jnp.sum(e, axis=-1, keepdims=True)
            attn = jnp.dot(e.astype(jnp.bfloat16), ctx2_ref[0:ns * L, :],
                           preferred_element_type=jnp.float32) * recip
            xwin = pltpu.roll(x_ref[pl.ds(a8, win), :], (win - d) % win,
                              axis=0)
            g = xwin[:nb] + jnp.dot(
                attn.astype(jnp.bfloat16), wa_ref[...],
                preferred_element_type=jnp.float32)
            zr = jax.nn.sigmoid(g[:, : 2 * H] + hm[:, ns * L:])
            z = zr[:, :H]
            r = zr[:, H:]
            n = jnp.tanh(g[:, 2 * H:] + jnp.dot(
                (r * h).astype(jnp.bfloat16), un_ref[...],
                preferred_element_type=jnp.float32))
            hn = (1.0 - z) * n + z * h
            hsel = jnp.where(lane < b, hn, h)
            owin = pltpu.roll(
                jnp.concatenate([hsel, jnp.zeros((8, H), jnp.float32)],
                                axis=0), d, axis=0)
            keep = (roww >= d) & (roww < d + nb)
            out_ref[pl.ds(a8, win), :] = jnp.where(
                keep, owin, out_ref[pl.ds(a8, win), :])
            return off + b, hsel

        def quad(it, carry):
            off, h = carry
            off, h = substep(4 * it, off, h)
            off, h = substep(4 * it + 1, off, h)
            off, h = substep(4 * it + 2, off, h)
            off, h = substep(4 * it + 3, off, h)
            return off, h

        return quad

    off = jnp.int32(0)
    h = jnp.zeros((B, H), jnp.float32)
    for p in range(4):
        off, h = jax.lax.fori_loop(16 * p, 16 * (p + 1),
                                   make_quad(16, 16 - 2 * p, hms[p]),
                                   (off, h))
    hf_ref[0, 8:, :] = h[8:]
    h = h[:8]
    for p in range(4, 8):
        off, h = jax.lax.fori_loop(16 * p, 16 * (p + 1),
                                   make_quad(8, 16 - 2 * p, hms[p]),
                                   (off, h))
    hf_ref[0, 0:8, :] = h


def kernel(pack_data, batch_sizes, context, context_mask, Wq, Wk, Wz, Wr, Wn,
           Uz, Ur, Un, bz, br, bn):
    f32 = jnp.float32
    pack_pad = jnp.zeros((PAD, D), f32).at[:TOTAL].set(pack_data)
    wcat = jnp.concatenate([Wz[:D], Wr[:D], Wn[:D]], axis=1)
    bcat = jnp.concatenate([bz, br, bn])[None, :]
    wa = jnp.concatenate([Wz[D:], Wr[D:], Wn[D:]], axis=1)
    ucat = jnp.concatenate([Uz, Ur], axis=1)
    ctx2 = context.reshape(B * L, CD)
    madd1 = jnp.where(context_mask, 0.0, -1e9).astype(f32)
    madd = jnp.where(jnp.eye(B, dtype=bool)[:, :, None],
                     madd1[:, None, :], -1e9).reshape(B, B * L).astype(f32)

    QKT = pl.pallas_call(
        _qkt_kernel,
        out_shape=jax.ShapeDtypeStruct((H, B * L), jnp.bfloat16),
    )(Wq, Wk, ctx2)
    ucat_b = ucat.astype(jnp.bfloat16)
    hms = [jnp.concatenate([QKT[:, : (16 - 2 * p) * L], ucat_b], axis=1)
           for p in range(8)]

    X = pl.pallas_call(
        _proj_kernel,
        out_shape=jax.ShapeDtypeStruct((PAD, 3 * H), f32),
        compiler_params=pltpu.CompilerParams(vmem_limit_bytes=100 * 2**20),
    )(pack_pad, wcat, bcat)

    out_pad, hidden_final = pl.pallas_call(
        _loop_kernel,
        out_shape=(jax.ShapeDtypeStruct((PAD, H), f32),
                   jax.ShapeDtypeStruct((1, B, H), f32)),
        compiler_params=pltpu.CompilerParams(vmem_limit_bytes=110 * 2**20),
    )(X, *hms, ctx2.astype(jnp.bfloat16), madd,
      Un.astype(jnp.bfloat16), wa.astype(jnp.bfloat16))

    return out_pad[:TOTAL], hidden_final
